# baseline (speedup 1.0000x reference)
import functools

import jax
import jax.numpy as jnp
from jax import lax
from jax.experimental import pallas as pl
from jax.experimental.pallas import tpu as pltpu
from jax.experimental.pallas import tpu_sc as plsc

_NC, _NS = 2, 16
_NW = _NC * _NS
_B = 4096
_F = 4
_E = 64
_N_IDX = _B * _F
_PER = _N_IDX // _NW
_CH = 64
_NCH = _PER // _CH
_ROWS_PER_W = _B // _NW
_ROWS_PER_CH = _CH // _F


def _sc_gather(xi, emb128, lin128):
    mesh = plsc.VectorSubcoreMesh(
        core_axis_name="c", subcore_axis_name="s",
        num_cores=_NC, num_subcores=_NS)

    @functools.partial(
        pl.kernel,
        out_type=(
            jax.ShapeDtypeStruct((_B, _F * _E), jnp.float32),
            jax.ShapeDtypeStruct((_NW, _PER), jnp.float32),
        ),
        mesh=mesh,
        scratch_types=[
            pltpu.VMEM((_NCH, _CH), jnp.int32),
            pltpu.VMEM((_NCH, _CH), jnp.int32),
            pltpu.VMEM((_NCH, _CH), jnp.int32),
            pltpu.VMEM((_CH, 128), jnp.float32),
            pltpu.VMEM((_CH, 128), jnp.float32),
            pltpu.VMEM((_CH, 128), jnp.float32),
            pltpu.VMEM((_CH, 128), jnp.float32),
            pltpu.VMEM((_ROWS_PER_CH, _F * _E), jnp.float32),
            pltpu.VMEM((_PER,), jnp.float32),
            pltpu.SemaphoreType.DMA,
            pltpu.SemaphoreType.DMA,
        ],
        compiler_params=pltpu.CompilerParams(
            use_tc_tiling_on_sc=True, needs_layout_passes=False),
    )
    def k(xi_hbm, emb_hbm, lin_hbm, emb_out, lin_out, idx_v, idxg_v, idxl_v,
          st0, st1, ln0, ln1, outst_v, lv_v, sem_e, sem_l):
        wid = lax.axis_index("s") * _NC + lax.axis_index("c")
        st = (st0, st1)
        ln = (ln0, ln1)
        pltpu.sync_copy(xi_hbm.at[wid], idx_v)
        for j in range(_NCH):
            for kk in range(_CH // 16):
                sl = pl.ds(kk * 16, 16)
                idxg_v[j, sl] = lax.shift_right_logical(idx_v[j, sl], 1)
                idxl_v[j, sl] = lax.shift_right_logical(idx_v[j, sl], 7)

        iota16 = lax.iota(jnp.int32, 16)

        def enqueue(j):
            return (
                pltpu.async_copy(
                    emb_hbm.at[idxg_v.at[j]], st[j % 2], sem_e),
                pltpu.async_copy(
                    lin_hbm.at[idxl_v.at[j]], ln[j % 2], sem_l),
            )

        pending = enqueue(0)
        for j in range(_NCH):
            cur = pending
            if j + 1 < _NCH:
                pending = enqueue(j + 1)
            for c in cur:
                c.wait()
            for g in range(_CH // 16):
                slots = jnp.full((16,), g * 16, jnp.int32) + iota16
                lanes = lax.bitwise_and(idx_v[j, pl.ds(g * 16, 16)], 127)
                lv_v[pl.ds(j * _CH + g * 16, 16)] = plsc.load_gather(
                    ln[j % 2], [slots, lanes])

            @pl.loop(0, _ROWS_PER_CH)
            def _(q):
                for f in range(_F):
                    pos = q * _F + f
                    pv = jnp.full((16,), pos, jnp.int32)
                    iv = plsc.load_gather(
                        idx_v, [jnp.full((16,), j, jnp.int32), pv])
                    base = lax.bitwise_and(iv, 1) * 64
                    for c in range(_E // 16):
                        val = plsc.load_gather(
                            st[j % 2], [pv, base + (c * 16) + iota16])
                        outst_v[q, pl.ds(f * _E + c * 16, 16)] = val

            pltpu.sync_copy(
                outst_v,
                emb_out.at[pl.ds(wid * _ROWS_PER_W + j * _ROWS_PER_CH,
                                 _ROWS_PER_CH)])

        pltpu.sync_copy(lv_v, lin_out.at[wid])

    return k(xi, emb128, lin128)


def _bn_relu(h, g, be):
    mu = jnp.mean(h, axis=0, keepdims=True)
    d = h - mu
    var = jnp.mean(d * d, axis=0, keepdims=True)
    return jnp.maximum(g * d * lax.rsqrt(var + 1e-5) + be, 0.0)


def _dense_body(emb_ref, lin_ref, w1_ref, b1_ref, g1_ref, be1_ref,
                w2_ref, b2_ref, g2_ref, be2_ref, w3_ref, b3_ref,
                bias_ref, out_ref):
    e = emb_ref[...]
    e0 = e[:, 0 * _E:1 * _E]
    e1 = e[:, 1 * _E:2 * _E]
    e2 = e[:, 2 * _E:3 * _E]
    e3 = e[:, 3 * _E:4 * _E]
    s = e0 + e1 + e2 + e3
    sq = s * s - (e0 * e0 + e1 * e1 + e2 * e2 + e3 * e3)
    fm = 0.5 * jnp.sum(sq, axis=1, keepdims=True)
    lin = jnp.sum(lin_ref[...], axis=1, keepdims=True) + bias_ref[0, 0]
    h = jnp.dot(e, w1_ref[...], preferred_element_type=jnp.float32)
    h = _bn_relu(h + b1_ref[...], g1_ref[...], be1_ref[...])
    h = jnp.dot(h, w2_ref[...], preferred_element_type=jnp.float32)
    h = _bn_relu(h + b2_ref[...], g2_ref[...], be2_ref[...])
    z = jnp.dot(h, w3_ref[...], preferred_element_type=jnp.float32)
    z = z + b3_ref[0, 0] + lin + fm
    out_ref[...] = 1.0 / (1.0 + jnp.exp(-z))


def _dense(emb, lin4, W1, b1, g1, be1, W2, b2, g2, be2, W3, b3, lin_bias,
           interpret=False):
    return pl.pallas_call(
        _dense_body,
        out_shape=jax.ShapeDtypeStruct((_B, 1), jnp.float32),
        interpret=interpret,
    )(emb, lin4, W1, b1.reshape(1, -1), g1.reshape(1, -1), be1.reshape(1, -1),
      W2, b2.reshape(1, -1), g2.reshape(1, -1), be2.reshape(1, -1),
      W3, b3.reshape(1, 1), lin_bias.reshape(1, 1))


def kernel(x, embed_table, lin_table, lin_bias, W1, b1, g1, be1,
           W2, b2, g2, be2, W3, b3):
    offsets = jnp.arange(_F, dtype=jnp.int32) * 100000
    xi = (x.astype(jnp.int32) + offsets[None, :]).reshape(_NW, _NCH, _CH)
    emb128 = embed_table.reshape(200000, 128)
    lin128 = lin_table.reshape(3125, 128)
    emb, lin_rows = _sc_gather(xi, emb128, lin128)
    lin4 = lin_rows.reshape(_B, _F)
    return _dense(emb, lin4, W1, b1, g1, be1, W2, b2, g2, be2, W3, b3,
                  lin_bias)


# speedup vs baseline: 1.0030x; 1.0030x over previous
import functools

import jax
import jax.numpy as jnp
from jax import lax
from jax.experimental import pallas as pl
from jax.experimental.pallas import tpu as pltpu
from jax.experimental.pallas import tpu_sc as plsc

_NC, _NS = 2, 16
_NW = _NC * _NS
_B = 4096
_F = 4
_E = 64
_N_IDX = _B * _F
_PER = _N_IDX // _NW
_CH = 64
_NCH = _PER // _CH


def _sc_gather(xi, emb128, lin128):
    mesh = plsc.VectorSubcoreMesh(
        core_axis_name="c", subcore_axis_name="s",
        num_cores=_NC, num_subcores=_NS)

    @functools.partial(
        pl.kernel,
        out_type=(
            jax.ShapeDtypeStruct((_N_IDX, 128), jnp.float32),
            jax.ShapeDtypeStruct((_NW, _PER), jnp.float32),
        ),
        mesh=mesh,
        scratch_types=[
            pltpu.VMEM((_NCH, _CH), jnp.int32),
            pltpu.VMEM((_NCH, _CH), jnp.int32),
            pltpu.VMEM((_NCH, _CH), jnp.int32),
            pltpu.VMEM((_CH, 128), jnp.float32),
            pltpu.VMEM((_CH, 128), jnp.float32),
            pltpu.VMEM((_CH, 128), jnp.float32),
            pltpu.VMEM((_CH, 128), jnp.float32),
            pltpu.VMEM((_PER,), jnp.float32),
            pltpu.SemaphoreType.DMA,
            pltpu.SemaphoreType.DMA,
        ],
        compiler_params=pltpu.CompilerParams(
            use_tc_tiling_on_sc=True, needs_layout_passes=False),
    )
    def k(xi_hbm, emb_hbm, lin_hbm, emb_out, lin_out, idx_v, idxg_v, idxl_v,
          st0, st1, ln0, ln1, lv_v, sem_e, sem_l):
        wid = lax.axis_index("s") * _NC + lax.axis_index("c")
        st = (st0, st1)
        ln = (ln0, ln1)
        pltpu.sync_copy(xi_hbm.at[wid], idx_v)
        for j in range(_NCH):
            for kk in range(_CH // 16):
                sl = pl.ds(kk * 16, 16)
                idxg_v[j, sl] = lax.shift_right_logical(idx_v[j, sl], 1)
                idxl_v[j, sl] = lax.shift_right_logical(idx_v[j, sl], 7)

        iota16 = lax.iota(jnp.int32, 16)

        def enqueue(j):
            return (
                pltpu.async_copy(
                    emb_hbm.at[idxg_v.at[j]], st[j % 2], sem_e),
                pltpu.async_copy(
                    lin_hbm.at[idxl_v.at[j]], ln[j % 2], sem_l),
            )

        pending = enqueue(0)
        for j in range(_NCH):
            cur = pending
            if j + 1 < _NCH:
                pending = enqueue(j + 1)
            for c in cur:
                c.wait()
            for g in range(_CH // 16):
                slots = jnp.full((16,), g * 16, jnp.int32) + iota16
                lanes = lax.bitwise_and(idx_v[j, pl.ds(g * 16, 16)], 127)
                lv_v[pl.ds(j * _CH + g * 16, 16)] = plsc.load_gather(
                    ln[j % 2], [slots, lanes])

            pltpu.sync_copy(
                st[j % 2],
                emb_out.at[pl.ds(wid * _PER + j * _CH, _CH)])

        pltpu.sync_copy(lv_v, lin_out.at[wid])

    return k(xi, emb128, lin128)


def _bn_relu(h, g, be):
    mu = jnp.mean(h, axis=0, keepdims=True)
    d = h - mu
    var = jnp.mean(d * d, axis=0, keepdims=True)
    return jnp.maximum(g * d * lax.rsqrt(var + 1e-5) + be, 0.0)


def _dense_body(emb_ref, par_ref, lin_ref, w1_ref, b1_ref, g1_ref, be1_ref,
                w2_ref, b2_ref, g2_ref, be2_ref, w3_ref, b3_ref,
                bias_ref, out_ref):
    es = []
    for f in range(_F):
        pair = emb_ref[f]
        pf = par_ref[:, f:f + 1]
        es.append(jnp.where(pf > 0.5, pair[:, _E:], pair[:, :_E]))
    e0, e1, e2, e3 = es
    s = e0 + e1 + e2 + e3
    sq = s * s - (e0 * e0 + e1 * e1 + e2 * e2 + e3 * e3)
    fm = 0.5 * jnp.sum(sq, axis=1, keepdims=True)
    e = jnp.concatenate(es, axis=1)
    lin = jnp.sum(lin_ref[...], axis=1, keepdims=True) + bias_ref[0, 0]
    h = jnp.dot(e, w1_ref[...], preferred_element_type=jnp.float32)
    h = _bn_relu(h + b1_ref[...], g1_ref[...], be1_ref[...])
    h = jnp.dot(h, w2_ref[...], preferred_element_type=jnp.float32)
    h = _bn_relu(h + b2_ref[...], g2_ref[...], be2_ref[...])
    z = jnp.dot(h, w3_ref[...], preferred_element_type=jnp.float32)
    z = z + b3_ref[0, 0] + lin + fm
    out_ref[...] = 1.0 / (1.0 + jnp.exp(-z))


def _dense(emb4, par, lin4, W1, b1, g1, be1, W2, b2, g2, be2, W3, b3,
           lin_bias, interpret=False):
    return pl.pallas_call(
        _dense_body,
        out_shape=jax.ShapeDtypeStruct((_B, 1), jnp.float32),
        interpret=interpret,
    )(emb4, par, lin4,
      W1, b1.reshape(1, -1), g1.reshape(1, -1), be1.reshape(1, -1),
      W2, b2.reshape(1, -1), g2.reshape(1, -1), be2.reshape(1, -1),
      W3, b3.reshape(1, 1), lin_bias.reshape(1, 1))


def kernel(x, embed_table, lin_table, lin_bias, W1, b1, g1, be1,
           W2, b2, g2, be2, W3, b3):
    offsets = jnp.arange(_F, dtype=jnp.int32) * 100000
    xio = x.astype(jnp.int32) + offsets[None, :]
    xi = xio.T.reshape(_NW, _NCH, _CH)
    par = lax.bitwise_and(xio, 1).astype(jnp.float32)
    emb128 = embed_table.reshape(200000, 128)
    lin128 = lin_table.reshape(3125, 128)
    emb_rows, lin_rows = _sc_gather(xi, emb128, lin128)
    emb4 = emb_rows.reshape(_F, _B, 128)
    lin4 = lin_rows.reshape(_F, _B).T
    return _dense(emb4, par, lin4, W1, b1, g1, be1, W2, b2, g2, be2, W3, b3,
                  lin_bias)


# speedup vs baseline: 1.1228x; 1.1194x over previous
import functools

import jax
import jax.numpy as jnp
from jax import lax
from jax.experimental import pallas as pl
from jax.experimental.pallas import tpu as pltpu
from jax.experimental.pallas import tpu_sc as plsc

_NC, _NS = 2, 16
_NW = _NC * _NS
_B = 4096
_F = 4
_E = 64
_N_IDX = _B * _F
_PER = _N_IDX // _NW
_CH = 64
_NCH = _PER // _CH


def _sc_gather(xi, emb128, lin128):
    mesh = plsc.VectorSubcoreMesh(
        core_axis_name="c", subcore_axis_name="s",
        num_cores=_NC, num_subcores=_NS)

    @functools.partial(
        pl.kernel,
        out_type=(
            jax.ShapeDtypeStruct((_N_IDX, 128), jnp.float32),
            jax.ShapeDtypeStruct((_NW, _PER), jnp.float32),
        ),
        mesh=mesh,
        scratch_types=[
            pltpu.VMEM((_NCH, _CH), jnp.int32),
            pltpu.VMEM((_NCH, _CH), jnp.int32),
            pltpu.VMEM((_CH, 128), jnp.float32),
            pltpu.VMEM((_CH, 128), jnp.float32),
            pltpu.VMEM((_CH, 128), jnp.float32),
            pltpu.VMEM((_CH, 128), jnp.float32),
            pltpu.VMEM((_PER,), jnp.float32),
            pltpu.SemaphoreType.DMA,
            pltpu.SemaphoreType.DMA,
        ],
        compiler_params=pltpu.CompilerParams(
            use_tc_tiling_on_sc=True, needs_layout_passes=False),
    )
    def k(xi_hbm, emb_hbm, lin_hbm, emb_out, lin_out, idx_v, idxl_v,
          st0, st1, ln0, ln1, lv_v, sem_e, sem_l):
        wid = lax.axis_index("s") * _NC + lax.axis_index("c")
        st = (st0, st1)
        ln = (ln0, ln1)
        pltpu.sync_copy(xi_hbm.at[wid], idx_v)
        for j in range(_NCH):
            for kk in range(_CH // 16):
                sl = pl.ds(kk * 16, 16)
                idxl_v[j, sl] = lax.shift_right_logical(idx_v[j, sl], 7)

        iota16 = lax.iota(jnp.int32, 16)

        def enqueue(j):
            return (
                pltpu.async_copy(
                    emb_hbm.at[idx_v.at[j]], st[j % 2], sem_e),
                pltpu.async_copy(
                    lin_hbm.at[idxl_v.at[j]], ln[j % 2], sem_l),
            )

        pending = enqueue(0)
        for j in range(_NCH):
            cur = pending
            if j + 1 < _NCH:
                pending = enqueue(j + 1)
            for c in cur:
                c.wait()
            for g in range(_CH // 16):
                slots = jnp.full((16,), g * 16, jnp.int32) + iota16
                lanes = lax.bitwise_and(idx_v[j, pl.ds(g * 16, 16)], 127)
                lv_v[pl.ds(j * _CH + g * 16, 16)] = plsc.load_gather(
                    ln[j % 2], [slots, lanes])

            pltpu.sync_copy(
                st[j % 2],
                emb_out.at[pl.ds(wid * _PER + j * _CH, _CH)])

        pltpu.sync_copy(lv_v, lin_out.at[wid])

    return k(xi, emb128, lin128)


def _bn_relu(h, g, be):
    mu = jnp.mean(h, axis=0, keepdims=True)
    d = h - mu
    var = jnp.mean(d * d, axis=0, keepdims=True)
    return jnp.maximum(g * d * lax.rsqrt(var + 1e-5) + be, 0.0)


def _dense_body(emb_ref, lin_ref, w1_ref, b1_ref, g1_ref, be1_ref,
                w2_ref, b2_ref, g2_ref, be2_ref, w3_ref, b3_ref,
                bias_ref, out_ref):
    es = [emb_ref[f][:, :_E] for f in range(_F)]
    e0, e1, e2, e3 = es
    s = e0 + e1 + e2 + e3
    sq = s * s - (e0 * e0 + e1 * e1 + e2 * e2 + e3 * e3)
    fm = 0.5 * jnp.sum(sq, axis=1, keepdims=True)
    e = jnp.concatenate(es, axis=1)
    lin = jnp.sum(lin_ref[...], axis=1, keepdims=True) + bias_ref[0, 0]
    h = jnp.dot(e, w1_ref[...], preferred_element_type=jnp.float32)
    h = _bn_relu(h + b1_ref[...], g1_ref[...], be1_ref[...])
    h = jnp.dot(h, w2_ref[...], preferred_element_type=jnp.float32)
    h = _bn_relu(h + b2_ref[...], g2_ref[...], be2_ref[...])
    z = jnp.dot(h, w3_ref[...], preferred_element_type=jnp.float32)
    z = z + b3_ref[0, 0] + lin + fm
    out_ref[...] = 1.0 / (1.0 + jnp.exp(-z))


def _dense(emb4, lin4, W1, b1, g1, be1, W2, b2, g2, be2, W3, b3,
           lin_bias, interpret=False):
    return pl.pallas_call(
        _dense_body,
        out_shape=jax.ShapeDtypeStruct((_B, 1), jnp.float32),
        interpret=interpret,
    )(emb4, lin4,
      W1, b1.reshape(1, -1), g1.reshape(1, -1), be1.reshape(1, -1),
      W2, b2.reshape(1, -1), g2.reshape(1, -1), be2.reshape(1, -1),
      W3, b3.reshape(1, 1), lin_bias.reshape(1, 1))


def kernel(x, embed_table, lin_table, lin_bias, W1, b1, g1, be1,
           W2, b2, g2, be2, W3, b3):
    offsets = jnp.arange(_F, dtype=jnp.int32) * 100000
    xio = x.astype(jnp.int32) + offsets[None, :]
    xi = xio.T.reshape(_NW, _NCH, _CH)
    emb128 = jnp.pad(
        embed_table.reshape(50000, 8, _E),
        ((0, 0), (0, 0), (0, 128 - _E))).reshape(400000, 128)
    lin128 = lin_table.reshape(3125, 128)
    emb_rows, lin_rows = _sc_gather(xi, emb128, lin128)
    emb4 = emb_rows.reshape(_F, _B, 128)
    lin4 = lin_rows.reshape(_F, _B).T
    return _dense(emb4, lin4, W1, b1, g1, be1, W2, b2, g2, be2, W3, b3,
                  lin_bias)
